# baseline (speedup 1.0000x reference)
import functools

import jax
import jax.numpy as jnp
from jax import lax
from jax.experimental import pallas as pl
from jax.experimental.pallas import tpu as pltpu
from jax.experimental.pallas import tpu_sc as plsc

_D = 64
_GROUP = 128
_K = 8
_CHUNK = _GROUP * _K

_info = plsc.get_sparse_core_info()
_NC = _info.num_cores
_NS = _info.num_subcores
_NW = _NC * _NS


def _make_lookup(n_rows):
    n_per_w = n_rows // _NW
    n_steps = n_per_w // _CHUNK
    mesh = plsc.VectorSubcoreMesh(core_axis_name="c", subcore_axis_name="s")

    @functools.partial(
        pl.kernel,
        mesh=mesh,
        out_type=jax.ShapeDtypeStruct((n_rows, _D), jnp.float32),
        scratch_types=[
            pltpu.VMEM((_K, _GROUP), jnp.int32),
            pltpu.VMEM((_CHUNK, _D), jnp.float32),
            pltpu.SemaphoreType.DMA,
        ],
        compiler_params=pltpu.CompilerParams(use_tc_tiling_on_sc=False),
    )
    def lookup(idx_hbm, table_hbm, out_hbm, idx_v, rows_v, sem):
        wid = lax.axis_index("s") * _NC + lax.axis_index("c")
        base = wid * n_per_w

        def step(g, _):
            off = base + g * _CHUNK
            row_off = pl.multiple_of(off // _GROUP, 8)
            pltpu.sync_copy(idx_hbm.at[pl.ds(row_off, _K), :], idx_v)
            copies = []
            for j in range(_K):
                copies.append(
                    pltpu.async_copy(
                        table_hbm.at[idx_v.at[j]],
                        rows_v.at[pl.ds(j * _GROUP, _GROUP)],
                        sem,
                    )
                )
            for c in copies:
                c.wait()
            pltpu.sync_copy(rows_v, out_hbm.at[pl.ds(off, _CHUNK)])
            return ()

        lax.fori_loop(0, n_steps, step, ())

    return lookup


def kernel(token_ids, weight):
    n_rows = token_ids.size
    idx2d = token_ids.reshape(n_rows // _GROUP, _GROUP)
    out = _make_lookup(n_rows)(idx2d, weight)
    return out.reshape(token_ids.shape + (weight.shape[1],))


# speedup vs baseline: 1.0172x; 1.0172x over previous
import functools

import jax
import jax.numpy as jnp
from jax import lax
from jax.experimental import pallas as pl
from jax.experimental.pallas import tpu as pltpu
from jax.experimental.pallas import tpu_sc as plsc

_D = 64
_GROUP = 128
_K = 5
_CHUNK = _GROUP * _K
_NBUF = 2

_info = plsc.get_sparse_core_info()
_NC = _info.num_cores
_NS = _info.num_subcores
_NW = _NC * _NS


def _make_lookup(n_rows):
    n_per_w = n_rows // _NW
    n_chunks = n_per_w // _CHUNK
    n_outer = n_chunks // _NBUF
    mesh = plsc.VectorSubcoreMesh(core_axis_name="c", subcore_axis_name="s")

    @functools.partial(
        pl.kernel,
        mesh=mesh,
        out_type=jax.ShapeDtypeStruct((n_rows, _D), jnp.float32),
        scratch_types=[
            pltpu.VMEM((n_per_w,), jnp.int32),
            pltpu.VMEM((_CHUNK, _D), jnp.float32),
            pltpu.VMEM((_CHUNK, _D), jnp.float32),
            pltpu.SemaphoreType.DMA,
            pltpu.SemaphoreType.DMA,
        ],
        compiler_params=pltpu.CompilerParams(use_tc_tiling_on_sc=False),
    )
    def lookup(idx_hbm, table_hbm, out_hbm, idx_v, rows0, rows1, g0, g1):
        rows = (rows0, rows1)
        gsem = (g0, g1)
        wid = lax.axis_index("s") * _NC + lax.axis_index("c")
        base = pl.multiple_of(wid * n_per_w, n_per_w)
        pltpu.sync_copy(idx_hbm.at[pl.ds(base, n_per_w)], idx_v)

        def fire_gathers(c, b):
            for j in range(_K):
                off = pl.multiple_of(c * _CHUNK + j * _GROUP, _GROUP)
                pltpu.async_copy(
                    table_hbm.at[idx_v.at[pl.ds(off, _GROUP)]],
                    rows[b].at[pl.ds(j * _GROUP, _GROUP)],
                    gsem[b],
                )

        for b in range(_NBUF):
            fire_gathers(jnp.int32(b), b)

        def outer(t, _):
            for b in range(_NBUF):
                c = t * _NBUF + b
                pltpu.make_async_copy(
                    out_hbm.at[pl.ds(0, _CHUNK)], rows[b], gsem[b]
                ).wait()
                off = pl.multiple_of(base + c * _CHUNK, _CHUNK)
                pltpu.sync_copy(rows[b], out_hbm.at[pl.ds(off, _CHUNK)])

                @pl.when(c + _NBUF < n_chunks)
                def _():
                    fire_gathers(c + _NBUF, b)

            return ()

        lax.fori_loop(0, n_outer, outer, ())

    return lookup


def kernel(token_ids, weight):
    n_rows = token_ids.size
    idx = token_ids.reshape(n_rows)
    out = _make_lookup(n_rows)(idx, weight)
    return out.reshape(token_ids.shape + (weight.shape[1],))
